# baseline (speedup 1.0000x reference)
import functools

import jax
import jax.numpy as jnp
from jax.experimental import pallas as pl
from jax.experimental.pallas import tpu as pltpu


def _bf(v):
    return v.astype(jnp.bfloat16)


def _bn_lrelu(t, g, beta, eps=1e-5, slope=0.01):
    m = jnp.mean(t, axis=0, keepdims=True)
    c = t - m
    v = jnp.mean(c * c, axis=0, keepdims=True)
    y = c * jax.lax.rsqrt(v + eps) * g + beta
    return jnp.where(y >= 0, y, slope * y)


def _gcn_body(adj_ref, x_ref, w0_ref, b0_ref, g0_ref, beta0_ref,
              w1_ref, b1_ref, t1_ref, t_scr, h1_scr, acc_scr, *, nb):
    s = pl.program_id(0)
    r, n = adj_ref.shape
    i = s % nb
    rows = jax.lax.broadcasted_iota(jnp.int32, (r, n), 0)
    cols = jax.lax.broadcasted_iota(jnp.int32, (r, n), 1)
    abf = _bf(adj_ref[...] + jnp.where(cols == rows + i * r, 1.0, 0.0))

    @pl.when(s < nb)
    def _layer1_mm():
        acc = jnp.dot(abf, x_ref[...], preferred_element_type=jnp.float32)
        t_scr[pl.ds(i * r, r), :] = (
            jnp.dot(_bf(acc), _bf(w0_ref[...]),
                    preferred_element_type=jnp.float32) + b0_ref[...])

    @pl.when(s == nb - 1)
    def _finish_layer1():
        h1_scr[...] = _bf(_bn_lrelu(t_scr[...], g0_ref[...], beta0_ref[...]))

    @pl.when(s >= nb)
    def _layer2_mm():
        acc = jnp.dot(abf, h1_scr[...], preferred_element_type=jnp.float32)
        t1_ref[...] = (
            jnp.dot(_bf(acc), _bf(w1_ref[...]),
                    preferred_element_type=jnp.float32) + b1_ref[...])

    del acc_scr


def _bn_lrelu_body(t_ref, g_ref, beta_ref, o_ref):
    o_ref[...] = _bn_lrelu(t_ref[...], g_ref[...], beta_ref[...])


def kernel(x, adj, W0, b0, g0, beta0, W1, b1, g1, beta1):
    n, d = x.shape
    r = 200 if n % 200 == 0 else n
    nb = n // r
    row2 = lambda v: v.reshape(1, -1)
    body = functools.partial(_gcn_body, nb=nb)
    t1 = pl.pallas_call(
        body,
        grid=(2 * nb,),
        in_specs=[
            pl.BlockSpec((r, n), lambda s: (s % nb, 0)),
            pl.BlockSpec((n, d), lambda s: (0, 0)),
            pl.BlockSpec((d, d), lambda s: (0, 0)),
            pl.BlockSpec((1, d), lambda s: (0, 0)),
            pl.BlockSpec((1, d), lambda s: (0, 0)),
            pl.BlockSpec((1, d), lambda s: (0, 0)),
            pl.BlockSpec((d, d), lambda s: (0, 0)),
            pl.BlockSpec((1, d), lambda s: (0, 0)),
        ],
        out_specs=pl.BlockSpec((r, d), lambda s: (s % nb, 0)),
        out_shape=jax.ShapeDtypeStruct((n, d), jnp.float32),
        scratch_shapes=[
            pltpu.VMEM((n, d), jnp.float32),
            pltpu.VMEM((n, d), jnp.bfloat16),
            pltpu.VMEM((r, d), jnp.float32),
        ],
        compiler_params=pltpu.CompilerParams(
            dimension_semantics=("arbitrary",),
        ),
    )(adj, _bf(x), W0, row2(b0), row2(g0), row2(beta0), W1, row2(b1))
    return pl.pallas_call(
        _bn_lrelu_body,
        in_specs=[
            pl.BlockSpec((n, d), lambda: (0, 0)),
            pl.BlockSpec((1, d), lambda: (0, 0)),
            pl.BlockSpec((1, d), lambda: (0, 0)),
        ],
        out_specs=pl.BlockSpec((n, d), lambda: (0, 0)),
        out_shape=jax.ShapeDtypeStruct((n, d), jnp.float32),
    )(t1, row2(g1), row2(beta1))


# speedup vs baseline: 1.3070x; 1.3070x over previous
import functools

import jax
import jax.numpy as jnp
from jax.experimental import pallas as pl
from jax.experimental.pallas import tpu as pltpu


def _bf(v):
    return v.astype(jnp.bfloat16)


def _layer_mm_body(adj_ref, h_ref, w_ref, b_ref, t_ref):
    i = pl.program_id(0)
    r, n = adj_ref.shape
    rows = jax.lax.broadcasted_iota(jnp.int32, (r, n), 0)
    cols = jax.lax.broadcasted_iota(jnp.int32, (r, n), 1)
    a = adj_ref[...] + jnp.where(cols == rows + i * r, 1.0, 0.0)
    acc = jnp.dot(_bf(a), _bf(h_ref[...]),
                  preferred_element_type=jnp.float32)
    t_ref[...] = jnp.dot(_bf(acc), _bf(w_ref[...]),
                         preferred_element_type=jnp.float32) + b_ref[...]


def _bn_lrelu_body(t_ref, g_ref, beta_ref, o_ref, *, eps, slope):
    t = t_ref[...]
    m = jnp.mean(t, axis=0, keepdims=True)
    c = t - m
    v = jnp.mean(c * c, axis=0, keepdims=True)
    y = c * jax.lax.rsqrt(v + eps) * g_ref[...] + beta_ref[...]
    o_ref[...] = jnp.where(y >= 0, y, slope * y)


def _layer_mm(adj, h, w, b, row_block):
    n, d = h.shape
    nb = n // row_block
    return pl.pallas_call(
        _layer_mm_body,
        grid=(nb,),
        in_specs=[
            pl.BlockSpec((row_block, n), lambda i: (i, 0)),
            pl.BlockSpec((n, d), lambda i: (0, 0)),
            pl.BlockSpec((d, d), lambda i: (0, 0)),
            pl.BlockSpec((1, d), lambda i: (0, 0)),
        ],
        out_specs=pl.BlockSpec((row_block, d), lambda i: (i, 0)),
        out_shape=jax.ShapeDtypeStruct((n, d), jnp.float32),
        compiler_params=pltpu.CompilerParams(
            dimension_semantics=("arbitrary",),
        ),
    )(adj, h, w, b)


def _bn_lrelu(t, g, beta):
    n, d = t.shape
    body = functools.partial(_bn_lrelu_body, eps=1e-5, slope=0.01)
    return pl.pallas_call(
        body,
        in_specs=[
            pl.BlockSpec((n, d), lambda: (0, 0)),
            pl.BlockSpec((1, d), lambda: (0, 0)),
            pl.BlockSpec((1, d), lambda: (0, 0)),
        ],
        out_specs=pl.BlockSpec((n, d), lambda: (0, 0)),
        out_shape=jax.ShapeDtypeStruct((n, d), jnp.float32),
    )(t, g, beta)


def kernel(x, adj, W0, b0, g0, beta0, W1, b1, g1, beta1):
    n = adj.shape[0]
    row_block = 400 if n % 400 == 0 else n
    h = x
    for (w, b, g, beta) in ((W0, b0, g0, beta0), (W1, b1, g1, beta1)):
        t = _layer_mm(adj, h, w, b.reshape(1, -1), row_block)
        h = _bn_lrelu(t, g.reshape(1, -1), beta.reshape(1, -1))
    return h
